# baseline (speedup 1.0000x reference)
import functools

import jax
import jax.numpy as jnp
from jax import lax
from jax.experimental import pallas as pl
from jax.experimental.pallas import tpu as pltpu
from jax.experimental.pallas import tpu_sc as plsc

_B, _C = 2, 96
_OH = _OW = 27
_KH = _KW = 16
_H = _W = 224
_N_IMG = _B * _C
_ROW_ELEMS = _OW * _KH * _KW
_IMG_OUT = _H * _W
_N_WORKERS = 32
_IMGS_PER_WORKER = _N_IMG // _N_WORKERS
_CHROWS = 3
_NCH = _OH // _CHROWS
_CHUNK = _CHROWS * _ROW_ELEMS


def _fold_sc(xr):
    mesh = plsc.VectorSubcoreMesh(core_axis_name="c", subcore_axis_name="s")

    @functools.partial(
        pl.kernel,
        out_type=jax.ShapeDtypeStruct((_N_IMG, _IMG_OUT), jnp.float32),
        mesh=mesh,
        scratch_types=[
            pltpu.VMEM((_CHUNK,), jnp.float32),
            pltpu.VMEM((_CHUNK,), jnp.float32),
            pltpu.VMEM((_CHUNK,), jnp.float32),
            pltpu.VMEM((_IMG_OUT,), jnp.float32),
            pltpu.SemaphoreType.DMA,
            pltpu.SemaphoreType.DMA,
            pltpu.SemaphoreType.DMA,
            pltpu.SemaphoreType.DMA,
        ],
    )
    def k(x_hbm, out_hbm, rb0, rb1, rb2, obuf, s0, s1, s2, so):
        wid = lax.axis_index("s") * 2 + lax.axis_index("c")
        zeros16 = jnp.zeros((16,), jnp.float32)
        sems = [s0, s1, s2]
        rbufs = [rb0, rb1, rb2]

        def wait_in(slot):
            pltpu.make_async_copy(x_hbm.at[0], rbufs[slot], sems[slot]).wait()

        def wait_out():
            pltpu.make_async_copy(obuf, out_hbm.at[0], so).wait()

        def compute_chunk(ch, slot):
            def kh_body(kh, carry):
                for r in range(_CHROWS):
                    dst_base = (8 * (ch * _CHROWS + r) + kh) * _W
                    src_base = r * _ROW_ELEMS + kh * _KW
                    vs = [
                        rbufs[slot][pl.ds(src_base + j * (_KH * _KW), 16)]
                        for j in range(_OW)
                    ]
                    for j in range(_OW):
                        plsc.addupdate(obuf.at[pl.ds(dst_base + 8 * j, 16)], vs[j])
                return carry

            lax.fori_loop(0, _KH, kh_body, 0)

        def zero_body(t, carry):
            for u in range(_W // 16):
                obuf[pl.ds(t * _W + u * 16, 16)] = zeros16
            return carry

        def image_body(m, carry):
            img = wid * _IMGS_PER_WORKER + m
            ibase = img * _NCH
            pltpu.async_copy(x_hbm.at[ibase], rb0, s0)
            pltpu.async_copy(x_hbm.at[ibase + 1], rb1, s1)

            @pl.when(m > 0)
            def _():
                wait_out()

            lax.fori_loop(0, _H, zero_body, 0)

            def g_body(g, inner):
                for s in range(3):
                    ch = 3 * g + s
                    wait_in(s)
                    compute_chunk(ch, s)
                    nslot = (s + 2) % 3

                    @pl.when(ch + 2 <= _NCH - 1)
                    def _():
                        pltpu.async_copy(
                            x_hbm.at[ibase + ch + 2], rbufs[nslot], sems[nslot]
                        )

                return inner

            lax.fori_loop(0, _NCH // 3, g_body, 0)
            pltpu.async_copy(obuf, out_hbm.at[img], so)
            return carry

        lax.fori_loop(0, _IMGS_PER_WORKER, image_body, 0)
        wait_out()

    return k(xr)


def kernel(x):
    xr = x.reshape(_N_IMG * _NCH, _CHUNK)
    out = _fold_sc(xr)
    return out.reshape(_B, _C, _H, _W)


# speedup vs baseline: 2.0320x; 2.0320x over previous
import functools

import jax
import jax.numpy as jnp
from jax import lax
from jax.experimental import pallas as pl
from jax.experimental.pallas import tpu as pltpu
from jax.experimental.pallas import tpu_sc as plsc

_B, _C = 2, 96
_OH = _OW = 27
_KH = _KW = 16
_H = _W = 224
_N_IMG = _B * _C
_ROW_ELEMS = _OW * _KH * _KW
_IMG_OUT = _H * _W
_N_WORKERS = 32
_IMGS_PER_WORKER = _N_IMG // _N_WORKERS


def _fold_sc(xr):
    mesh = plsc.VectorSubcoreMesh(core_axis_name="c", subcore_axis_name="s")

    @functools.partial(
        pl.kernel,
        out_type=jax.ShapeDtypeStruct((_N_IMG, _IMG_OUT), jnp.float32),
        mesh=mesh,
        scratch_types=[
            pltpu.VMEM((_ROW_ELEMS,), jnp.float32),
            pltpu.VMEM((_IMG_OUT,), jnp.float32),
        ],
    )
    def k(x_hbm, out_hbm, xrow, obuf):
        wid = lax.axis_index("s") * 2 + lax.axis_index("c")
        zeros16 = jnp.zeros((16,), jnp.float32)

        def zero_body(t, carry):
            for u in range(_W // 16):
                obuf[pl.ds(t * _W + u * 16, 16)] = zeros16
            return carry

        def run_image(img):
            lax.fori_loop(0, _H, zero_body, 0)

            def row_body(i, carry):
                pltpu.sync_copy(x_hbm.at[img, i], xrow)

                def kh_body(kh, inner):
                    base_dst = (8 * i + kh) * _W
                    base_src = kh * _KW
                    vs = [
                        xrow[pl.ds(base_src + j * (_KH * _KW), 16)]
                        for j in range(_OW)
                    ]
                    for j in range(_OW):
                        plsc.addupdate(obuf.at[pl.ds(base_dst + 8 * j, 16)], vs[j])
                    return inner

                lax.fori_loop(0, _KH, kh_body, 0)
                return carry

            lax.fori_loop(0, _OH, row_body, 0)
            pltpu.sync_copy(obuf, out_hbm.at[img])

        for m in range(_IMGS_PER_WORKER):
            run_image(wid * _IMGS_PER_WORKER + m)

    return k(xr)


def kernel(x):
    xr = x.reshape(_N_IMG, _OH, _ROW_ELEMS)
    out = _fold_sc(xr)
    return out.reshape(_B, _C, _H, _W)


# speedup vs baseline: 2.4833x; 1.2221x over previous
import functools

import jax
import jax.numpy as jnp
from jax import lax
from jax.experimental import pallas as pl
from jax.experimental.pallas import tpu as pltpu
from jax.experimental.pallas import tpu_sc as plsc

_B, _C = 2, 96
_OH = _OW = 27
_KH = _KW = 16
_H = _W = 224
_N_IMG = _B * _C
_ROW_ELEMS = _OW * _KH * _KW
_IMG_OUT = _H * _W
_N_WORKERS = 32
_IMGS_PER_WORKER = _N_IMG // _N_WORKERS


def _fold_sc(xr):
    mesh = plsc.VectorSubcoreMesh(core_axis_name="c", subcore_axis_name="s")

    @functools.partial(
        pl.kernel,
        out_type=jax.ShapeDtypeStruct((_N_IMG, _IMG_OUT), jnp.float32),
        mesh=mesh,
        scratch_types=[
            pltpu.VMEM((_ROW_ELEMS,), jnp.float32),
            pltpu.VMEM((_ROW_ELEMS,), jnp.float32),
            pltpu.VMEM((_ROW_ELEMS,), jnp.float32),
            pltpu.VMEM((_IMG_OUT,), jnp.float32),
            pltpu.VMEM((_IMG_OUT,), jnp.float32),
            pltpu.SemaphoreType.DMA,
            pltpu.SemaphoreType.DMA,
            pltpu.SemaphoreType.DMA,
            pltpu.SemaphoreType.DMA,
            pltpu.SemaphoreType.DMA,
        ],
    )
    def k(x_hbm, out_hbm, rb0, rb1, rb2, obA, obB, s0, s1, s2, soA, soB):
        wid = lax.axis_index("s") * 2 + lax.axis_index("c")
        zeros16 = jnp.zeros((16,), jnp.float32)
        rbufs = [rb0, rb1, rb2]
        sems = [s0, s1, s2]
        obufs = [obA, obB]
        osems = [soA, soB]

        def wait_in(slot):
            pltpu.make_async_copy(x_hbm.at[0, 0], rbufs[slot], sems[slot]).wait()

        def wait_out(p):
            pltpu.make_async_copy(obufs[p], out_hbm.at[0], osems[p]).wait()

        def run_image(img, p, first):
            obuf = obufs[p]
            pltpu.async_copy(x_hbm.at[img, 0], rb0, s0)
            pltpu.async_copy(x_hbm.at[img, 1], rb1, s1)
            if not first:
                wait_out(p)

            def zero_body(t, carry):
                for u in range(_W // 16):
                    obuf[pl.ds(t * _W + u * 16, 16)] = zeros16
                return carry

            lax.fori_loop(0, _H, zero_body, 0)

            def g_body(g, carry):
                for s in range(3):
                    i = 3 * g + s
                    wait_in(s)
                    nslot = (s + 2) % 3

                    @pl.when(i + 2 <= _OH - 1)
                    def _():
                        pltpu.async_copy(
                            x_hbm.at[img, i + 2], rbufs[nslot], sems[nslot]
                        )

                    def kh_body(kh, inner):
                        base_dst = (8 * i + kh) * _W
                        base_src = kh * _KW
                        vs = [
                            rbufs[s][pl.ds(base_src + j * (_KH * _KW), 16)]
                            for j in range(_OW)
                        ]
                        for j in range(_OW):
                            plsc.addupdate(
                                obuf.at[pl.ds(base_dst + 8 * j, 16)], vs[j]
                            )
                        return inner

                    lax.fori_loop(0, _KH, kh_body, 0)
                return carry

            lax.fori_loop(0, _OH // 3, g_body, 0)
            pltpu.async_copy(obuf, out_hbm.at[img], osems[p])

        for m in range(_IMGS_PER_WORKER):
            run_image(wid * _IMGS_PER_WORKER + m, m % 2, m < 2)
        wait_out(0)
        wait_out(1)

    return k(xr)


def kernel(x):
    xr = x.reshape(_N_IMG, _OH, _ROW_ELEMS)
    out = _fold_sc(xr)
    return out.reshape(_B, _C, _H, _W)
